# Initial kernel scaffold; baseline (speedup 1.0000x reference)
import jax
import jax.numpy as jnp
from jax.experimental import pallas as pl


def kernel(x0, x1, b_idxes, i_idxes, j_idxes):
    raise NotImplementedError("write your pallas kernel here")


# speedup vs baseline: 5.4441x; 5.4441x over previous
import functools

import jax
import jax.numpy as jnp
from jax import lax
from jax.experimental import pallas as pl
from jax.experimental.pallas import tpu as pltpu
from jax.experimental.pallas import tpu_sc as plsc

W_SIZE = 5
STRIDE = 4
PADDING = 2
RIGHT_EXTRA = 1

_NW = 32
_CH = 128
_LANES = 16


def _build_sc_gather(M, C, Hp0, Wp0, Hp1, Wp1, ow, w0, w1):
    k0 = w0 * w0
    k1 = w1 * w1
    mpw = -(-M // _NW)
    mpw = -(-mpw // 8) * 8
    mpad = mpw * _NW
    r0 = mpw * k0
    r1 = mpw * k1
    nf0, t0 = divmod(r0, _CH)
    nf1, t1 = divmod(r1, _CH)
    groups = mpw // _LANES

    assert ow & (ow - 1) == 0
    ow_shift = ow.bit_length() - 1

    mesh = plsc.VectorSubcoreMesh(core_axis_name="c", subcore_axis_name="s")

    @functools.partial(
        pl.kernel,
        mesh=mesh,
        compiler_params=pltpu.CompilerParams(needs_layout_passes=False),
        out_type=(
            jax.ShapeDtypeStruct((mpad * k0, C), jnp.float32),
            jax.ShapeDtypeStruct((mpad * k1, C), jnp.float32),
        ),
        scratch_types=[
            pltpu.VMEM((mpw,), jnp.int32),
            pltpu.VMEM((mpw,), jnp.int32),
            pltpu.VMEM((mpw,), jnp.int32),
            pltpu.VMEM(((nf0 + 1) * _CH,), jnp.int32),
            pltpu.VMEM(((nf1 + 1) * _CH,), jnp.int32),
            pltpu.VMEM((_CH, C), jnp.float32),
            pltpu.SemaphoreType.DMA,
        ],
    )
    def sc_fn(tab0, tab1, b_hbm, i_hbm, j_hbm, out0, out1,
              b_v, i_v, j_v, idx0_v, idx1_v, rows_v, sem):
        wid = lax.axis_index("s") * 2 + lax.axis_index("c")
        mbase = wid * mpw
        pltpu.sync_copy(b_hbm.at[pl.ds(mbase, mpw)], b_v)
        pltpu.sync_copy(i_hbm.at[pl.ds(mbase, mpw)], i_v)
        pltpu.sync_copy(j_hbm.at[pl.ds(mbase, mpw)], j_v)

        zeros = jnp.zeros((_LANES,), jnp.int32)
        for s_ in range(0, _CH, _LANES):
            idx0_v[pl.ds(nf0 * _CH + s_, _LANES)] = zeros
            idx1_v[pl.ds(nf1 * _CH + s_, _LANES)] = zeros

        lanes = lax.iota(jnp.int32, _LANES)

        def fill(g, carry):
            m0 = g * _LANES
            bv = b_v[pl.ds(m0, _LANES)]
            iv = i_v[pl.ds(m0, _LANES)]
            jv = j_v[pl.ds(m0, _LANES)]
            base0 = (bv * (Hp0 * Wp0)
                     + (iv >> ow_shift) * (STRIDE * Wp0)
                     + (iv & (ow - 1)) * STRIDE)
            base1 = (bv * (Hp1 * Wp1)
                     + (jv >> ow_shift) * (STRIDE * Wp1)
                     + (jv & (ow - 1)) * STRIDE)
            ml = m0 + lanes
            for k in range(k0):
                plsc.store_scatter(idx0_v, [ml * k0 + k],
                                   base0 + (k // w0) * Wp0 + (k % w0))
            for k in range(k1):
                plsc.store_scatter(idx1_v, [ml * k1 + k],
                                   base1 + (k // w1) * Wp1 + (k % w1))
            return carry

        lax.fori_loop(0, groups, fill, 0)

        def gather_all(tab, idx_v, out, obase, nfull, tail):
            def chunk(c, carry):
                off = pl.multiple_of(c * _CH, _CH)
                pltpu.async_copy(tab.at[idx_v.at[pl.ds(off, _CH)]],
                                 rows_v, sem).wait()
                pltpu.sync_copy(rows_v, out.at[pl.ds(obase + c * _CH, _CH)])
                return carry
            lax.fori_loop(0, nfull, chunk, 0)
            pltpu.async_copy(tab.at[idx_v.at[pl.ds(nfull * _CH, _CH)]],
                             rows_v, sem).wait()
            pltpu.sync_copy(rows_v.at[pl.ds(0, tail)],
                            out.at[pl.ds(obase + nfull * _CH, tail)])

        gather_all(tab0, idx0_v, out0, wid * r0, nf0, t0)
        gather_all(tab1, idx1_v, out1, wid * r1, nf1, t1)

    return mpad, sc_fn


def kernel(x0, x1, b_idxes, i_idxes, j_idxes):
    w0 = W_SIZE
    e = RIGHT_EXTRA
    w1 = w0 + 2 * e
    p0 = PADDING
    p1 = PADDING + e
    N, C, H, W = x0.shape
    ow = (W + 2 * p0 - w0) // STRIDE + 1
    Hp0, Wp0 = H + 2 * p0, W + 2 * p0
    Hp1, Wp1 = H + 2 * p1, W + 2 * p1
    M = b_idxes.shape[0]

    t0 = jnp.pad(jnp.transpose(x0, (0, 2, 3, 1)),
                 ((0, 0), (p0, p0), (p0, p0), (0, 0)))
    t0 = t0.reshape(N * Hp0 * Wp0, C)
    t1 = jnp.pad(jnp.transpose(x1, (0, 2, 3, 1)),
                 ((0, 0), (p1, p1), (p1, p1), (0, 0)))
    t1 = t1.reshape(N * Hp1 * Wp1, C)

    mpad, sc_fn = _build_sc_gather(M, C, Hp0, Wp0, Hp1, Wp1, ow, w0, w1)
    pad = mpad - M
    b = jnp.pad(b_idxes.astype(jnp.int32), (0, pad))
    ii = jnp.pad(i_idxes.astype(jnp.int32), (0, pad))
    jj = jnp.pad(j_idxes.astype(jnp.int32), (0, pad))

    out0f, out1f = sc_fn(t0, t1, b, ii, jj)
    out0 = out0f.reshape(mpad, w0 * w0, C)[:M]
    out1 = out1f.reshape(mpad, w1 * w1, C)[:M]
    return out0, out1


# speedup vs baseline: 5.6945x; 1.0460x over previous
import functools

import jax
import jax.numpy as jnp
from jax import lax
from jax.experimental import pallas as pl
from jax.experimental.pallas import tpu as pltpu
from jax.experimental.pallas import tpu_sc as plsc

W_SIZE = 5
STRIDE = 4
PADDING = 2
RIGHT_EXTRA = 1

_NW = 32
_CH = 128
_LANES = 16


def _build_sc_gather(M, C, Hp0, Wp0, Hp1, Wp1, ow, w0, w1):
    k0 = w0 * w0
    k1 = w1 * w1
    mpw = -(-M // _NW)
    mpw = -(-mpw // 8) * 8
    mpad = mpw * _NW
    r0 = mpw * k0
    r1 = mpw * k1
    nf0, t0 = divmod(r0, _CH)
    nf1, t1 = divmod(r1, _CH)
    groups = mpw // _LANES

    assert ow & (ow - 1) == 0
    ow_shift = ow.bit_length() - 1

    mesh = plsc.VectorSubcoreMesh(core_axis_name="c", subcore_axis_name="s")

    @functools.partial(
        pl.kernel,
        mesh=mesh,
        compiler_params=pltpu.CompilerParams(needs_layout_passes=False),
        out_type=(
            jax.ShapeDtypeStruct((mpad * k0, C), jnp.float32),
            jax.ShapeDtypeStruct((mpad * k1, C), jnp.float32),
        ),
        scratch_types=[
            pltpu.VMEM((mpw,), jnp.int32),
            pltpu.VMEM((mpw,), jnp.int32),
            pltpu.VMEM((mpw,), jnp.int32),
            pltpu.VMEM(((nf0 + 1) * _CH,), jnp.int32),
            pltpu.VMEM(((nf1 + 1) * _CH,), jnp.int32),
            pltpu.VMEM((_CH, C), jnp.float32),
            pltpu.VMEM((_CH, C), jnp.float32),
            pltpu.VMEM((_CH, C), jnp.float32),
            pltpu.VMEM((_CH, C), jnp.float32),
            pltpu.SemaphoreType.DMA,
            pltpu.SemaphoreType.DMA,
            pltpu.SemaphoreType.DMA,
            pltpu.SemaphoreType.DMA,
        ],
    )
    def sc_fn(tab0, tab1, b_hbm, i_hbm, j_hbm, out0, out1,
              b_v, i_v, j_v, idx0_v, idx1_v,
              rows0, rows1, rows2, rows3, sem0, sem1, sem2, sem3):
        rows = (rows0, rows1, rows2, rows3)
        sems = (sem0, sem1, sem2, sem3)
        wid = lax.axis_index("s") * 2 + lax.axis_index("c")
        mbase = wid * mpw
        pltpu.sync_copy(b_hbm.at[pl.ds(mbase, mpw)], b_v)
        pltpu.sync_copy(i_hbm.at[pl.ds(mbase, mpw)], i_v)
        pltpu.sync_copy(j_hbm.at[pl.ds(mbase, mpw)], j_v)

        zeros = jnp.zeros((_LANES,), jnp.int32)
        for s_ in range(0, _CH, _LANES):
            idx0_v[pl.ds(nf0 * _CH + s_, _LANES)] = zeros
            idx1_v[pl.ds(nf1 * _CH + s_, _LANES)] = zeros

        lanes = lax.iota(jnp.int32, _LANES)

        def fill(g, carry):
            m0 = g * _LANES
            bv = b_v[pl.ds(m0, _LANES)]
            iv = i_v[pl.ds(m0, _LANES)]
            jv = j_v[pl.ds(m0, _LANES)]
            base0 = (bv * (Hp0 * Wp0)
                     + (iv >> ow_shift) * (STRIDE * Wp0)
                     + (iv & (ow - 1)) * STRIDE)
            base1 = (bv * (Hp1 * Wp1)
                     + (jv >> ow_shift) * (STRIDE * Wp1)
                     + (jv & (ow - 1)) * STRIDE)
            ml = m0 + lanes
            for k in range(k0):
                plsc.store_scatter(idx0_v, [ml * k0 + k],
                                   base0 + (k // w0) * Wp0 + (k % w0))
            for k in range(k1):
                plsc.store_scatter(idx1_v, [ml * k1 + k],
                                   base1 + (k // w1) * Wp1 + (k % w1))
            return carry

        lax.fori_loop(0, groups, fill, 0)

        def gather_all(tab, idx_v, out, obase, nfull, tail, pending):
            def wait64(b):
                pltpu.make_async_copy(out.at[pl.ds(obase, _CH)],
                                      rows[b], sems[b]).wait()

            def fire_gather(c, b):
                off = pl.multiple_of(c * _CH, _CH)
                pltpu.async_copy(tab.at[idx_v.at[pl.ds(off, _CH)]],
                                 rows[b], sems[b])

            def fire_write(c, b):
                pltpu.async_copy(rows[b],
                                 out.at[pl.ds(obase + c * _CH, _CH)],
                                 sems[b])

            for b in range(4):
                if pending[b]:
                    wait64(b)
                fire_gather(b, b)

            nq, rem = divmod(nfull, 4)

            def quad(q, carry):
                for b in range(4):
                    c = q * 4 + b
                    wait64(b)
                    fire_write(c, b)

                    @pl.when(c + 4 < nfull)
                    def _():
                        wait64(b)
                        fire_gather(c + 4, b)
                return carry

            lax.fori_loop(0, nq, quad, 0)
            for c in range(4 * nq, nfull):
                b = c % 4
                wait64(b)
                fire_write(c, b)
            wait64(0)
            fire_gather(nfull, 0)
            wait64(0)
            pltpu.sync_copy(rows[0].at[pl.ds(0, tail)],
                            out.at[pl.ds(obase + nfull * _CH, tail)])
            return (False, True, True, True)

        pend = gather_all(tab0, idx0_v, out0, wid * r0, nf0, t0,
                          (False,) * 4)
        pend = gather_all(tab1, idx1_v, out1, wid * r1, nf1, t1, pend)
        for b in range(4):
            if pend[b]:
                pltpu.make_async_copy(out1.at[pl.ds(wid * r1, _CH)],
                                      rows[b], sems[b]).wait()

    return mpad, sc_fn


def kernel(x0, x1, b_idxes, i_idxes, j_idxes):
    w0 = W_SIZE
    e = RIGHT_EXTRA
    w1 = w0 + 2 * e
    p0 = PADDING
    p1 = PADDING + e
    N, C, H, W = x0.shape
    ow = (W + 2 * p0 - w0) // STRIDE + 1
    Hp0, Wp0 = H + 2 * p0, W + 2 * p0
    Hp1, Wp1 = H + 2 * p1, W + 2 * p1
    M = b_idxes.shape[0]

    t0 = jnp.pad(jnp.transpose(x0, (0, 2, 3, 1)),
                 ((0, 0), (p0, p0), (p0, p0), (0, 0)))
    t0 = t0.reshape(N * Hp0 * Wp0, C)
    t1 = jnp.pad(jnp.transpose(x1, (0, 2, 3, 1)),
                 ((0, 0), (p1, p1), (p1, p1), (0, 0)))
    t1 = t1.reshape(N * Hp1 * Wp1, C)

    mpad, sc_fn = _build_sc_gather(M, C, Hp0, Wp0, Hp1, Wp1, ow, w0, w1)
    pad = mpad - M
    b = jnp.pad(b_idxes.astype(jnp.int32), (0, pad))
    ii = jnp.pad(i_idxes.astype(jnp.int32), (0, pad))
    jj = jnp.pad(j_idxes.astype(jnp.int32), (0, pad))

    out0f, out1f = sc_fn(t0, t1, b, ii, jj)
    out0 = out0f.reshape(mpad, w0 * w0, C)[:M]
    out1 = out1f.reshape(mpad, w1 * w1, C)[:M]
    return out0, out1
